# Initial kernel scaffold; baseline (speedup 1.0000x reference)
import jax
import jax.numpy as jnp
from jax.experimental import pallas as pl


def kernel(x, table):
    raise NotImplementedError("write your pallas kernel here")


# speedup vs baseline: 1.8716x; 1.8716x over previous
import functools

import jax
import jax.numpy as jnp
from jax import lax
from jax.experimental import pallas as pl
from jax.experimental.pallas import tpu as pltpu
from jax.experimental.pallas import tpu_sc as plsc

D = 64
B = 16384 * 50
NW = 32
BPW = B // NW
C = 800
NCH = BPW // C

_mesh = plsc.VectorSubcoreMesh(core_axis_name="c", subcore_axis_name="s")


@functools.partial(
    pl.kernel,
    out_type=jax.ShapeDtypeStruct((B, D), jnp.float32),
    mesh=_mesh,
    compiler_params=pltpu.CompilerParams(use_tc_tiling_on_sc=False),
    scratch_types=[
        pltpu.VMEM((BPW,), jnp.int32),
        pltpu.VMEM((C, D), jnp.float32),
        pltpu.VMEM((C, D), jnp.float32),
        pltpu.SemaphoreType.DMA,
        pltpu.SemaphoreType.DMA,
        pltpu.SemaphoreType.DMA,
        pltpu.SemaphoreType.DMA,
    ],
)
def _embedding_gather(idx_hbm, table_hbm, out_hbm,
                      idx_v, rows0, rows1, g0, g1, w0, w1):
    wid = lax.axis_index("s") * 2 + lax.axis_index("c")
    base = wid * BPW
    rows = (rows0, rows1)
    gsem = (g0, g1)
    wsem = (w0, w1)

    pltpu.sync_copy(idx_hbm.at[pl.ds(base, BPW)], idx_v)

    def gather_start(j, p):
        pltpu.make_async_copy(
            table_hbm.at[idx_v.at[pl.ds(j * C, C)]], rows[p], gsem[p]
        ).start()

    def gather_wait(p):
        pltpu.make_async_copy(
            table_hbm.at[idx_v.at[pl.ds(0, C)]], rows[p], gsem[p]
        ).wait()

    def write_start(j, p):
        pltpu.make_async_copy(
            rows[p], out_hbm.at[pl.ds(base + j * C, C)], wsem[p]
        ).start()

    def write_wait(p):
        pltpu.make_async_copy(
            rows[p], out_hbm.at[pl.ds(base, C)], wsem[p]
        ).wait()

    gather_start(0, 0)

    @pl.loop(0, NCH, step=2)
    def _chunks(j0):
        for b in range(2):
            j = j0 + b
            p = b
            gather_wait(p)

            @pl.when(j >= 1)
            def _():
                write_wait(1 - p)

            @pl.when(j + 1 < NCH)
            def _():
                gather_start(j + 1, 1 - p)

            write_start(j, p)

    write_wait((NCH - 1) % 2)


def kernel(x, table):
    flat = x.reshape(-1).astype(jnp.int32)
    out = _embedding_gather(flat, table)
    return out.reshape(x.shape + (D,))


# speedup vs baseline: 1.8743x; 1.0014x over previous
import functools

import jax
import jax.numpy as jnp
from jax import lax
from jax.experimental import pallas as pl
from jax.experimental.pallas import tpu as pltpu
from jax.experimental.pallas import tpu_sc as plsc

D = 64
B = 16384 * 50
NW = 32
BPW = B // NW
C = 400
NBUF = 4
NCH = BPW // C

_mesh = plsc.VectorSubcoreMesh(core_axis_name="c", subcore_axis_name="s")


@functools.partial(
    pl.kernel,
    out_type=jax.ShapeDtypeStruct((B, D), jnp.float32),
    mesh=_mesh,
    compiler_params=pltpu.CompilerParams(use_tc_tiling_on_sc=False),
    scratch_types=(
        [pltpu.VMEM((BPW,), jnp.int32)]
        + [pltpu.VMEM((C, D), jnp.float32)] * NBUF
        + [pltpu.SemaphoreType.DMA] * NBUF
        + [pltpu.SemaphoreType.DMA] * NBUF
    ),
)
def _embedding_gather(idx_hbm, table_hbm, out_hbm, idx_v, *bufs):
    rows = bufs[:NBUF]
    gsem = bufs[NBUF:2 * NBUF]
    wsem = bufs[2 * NBUF:]
    wid = lax.axis_index("s") * 2 + lax.axis_index("c")
    base = wid * BPW

    pltpu.sync_copy(idx_hbm.at[pl.ds(base, BPW)], idx_v)

    def gather_start(j, p):
        pltpu.make_async_copy(
            table_hbm.at[idx_v.at[pl.ds(j * C, C)]], rows[p], gsem[p]
        ).start()

    def gather_wait(p):
        pltpu.make_async_copy(
            table_hbm.at[idx_v.at[pl.ds(0, C)]], rows[p], gsem[p]
        ).wait()

    def write_start(j, p):
        pltpu.make_async_copy(
            rows[p], out_hbm.at[pl.ds(base + j * C, C)], wsem[p]
        ).start()

    def write_wait(p):
        pltpu.make_async_copy(
            rows[p], out_hbm.at[pl.ds(base, C)], wsem[p]
        ).wait()

    for j in range(NBUF - 1):
        gather_start(j, j)

    @pl.loop(0, NCH, step=NBUF)
    def _chunks(j0):
        for b in range(NBUF):
            j = j0 + b
            p = b
            gather_wait(p)

            q = (b - 1) % NBUF

            @pl.when(j >= 1)
            def _():
                write_wait(q)

            @pl.when(j + NBUF - 1 < NCH)
            def _():
                gather_start(j + NBUF - 1, q)

            write_start(j, p)

    write_wait((NCH - 1) % NBUF)


def kernel(x, table):
    flat = x.reshape(-1).astype(jnp.int32)
    out = _embedding_gather(flat, table)
    return out.reshape(x.shape + (D,))


# speedup vs baseline: 1.9594x; 1.0454x over previous
import functools

import jax
import jax.numpy as jnp
from jax import lax
from jax.experimental import pallas as pl
from jax.experimental.pallas import tpu as pltpu
from jax.experimental.pallas import tpu_sc as plsc

D = 64
B = 16384 * 50
NW = 32
BPW = B // NW
C = 400
NBUF = 4
NCH = BPW // C

_mesh = plsc.VectorSubcoreMesh(core_axis_name="c", subcore_axis_name="s")


@functools.partial(
    pl.kernel,
    out_type=jax.ShapeDtypeStruct((B, D), jnp.float32),
    mesh=_mesh,
    compiler_params=pltpu.CompilerParams(use_tc_tiling_on_sc=False),
    scratch_types=(
        [pltpu.VMEM((BPW,), jnp.int32)]
        + [pltpu.VMEM((C, D), jnp.float32)] * NBUF
        + [pltpu.SemaphoreType.DMA] * NBUF
        + [pltpu.SemaphoreType.DMA] * NBUF
    ),
)
def _embedding_gather(idx_hbm, table_hbm, out_hbm, idx_v, *bufs):
    rows = bufs[:NBUF]
    gsem = bufs[NBUF:2 * NBUF]
    wsem = bufs[2 * NBUF:]
    wid = lax.axis_index("s") * 2 + lax.axis_index("c")
    base = wid * BPW

    pltpu.sync_copy(idx_hbm.at[pl.ds(base, BPW)], idx_v)

    def gather_start(j, p):
        pltpu.make_async_copy(
            table_hbm.at[idx_v.at[pl.ds(j * C, C)]], rows[p], gsem[p]
        ).start()

    def gather_wait(p):
        pltpu.make_async_copy(
            table_hbm.at[idx_v.at[pl.ds(0, C)]], rows[p], gsem[p]
        ).wait()

    def write_start(j, p):
        pltpu.make_async_copy(
            rows[p], out_hbm.at[pl.ds(base + j * C, C)], wsem[p]
        ).start()

    def write_wait(p):
        pltpu.make_async_copy(
            rows[p], out_hbm.at[pl.ds(base, C)], wsem[p]
        ).wait()

    for j in range(NBUF - 1):
        gather_start(j, j)

    @pl.loop(0, NCH, step=NBUF)
    def _chunks(j0):
        for b in range(NBUF):
            j = j0 + b
            p = b
            gather_wait(p)

            q = (b - 1) % NBUF

            @pl.when(j >= 1)
            def _():
                write_wait(q)

            @pl.when(j + NBUF - 1 < NCH)
            def _():
                gather_start(j + NBUF - 1, q)

            write_start(j, p)

    write_wait((NCH - 1) % NBUF)


def kernel(x, table):
    n, s = x.shape
    flat = x.T.reshape(-1).astype(jnp.int32)
    out = _embedding_gather(flat, table)
    return out.reshape(s, n, D).transpose(1, 0, 2)
